# baseline (speedup 1.0000x reference)
import functools

import jax
import jax.numpy as jnp
from jax import lax
from jax.experimental import pallas as pl
from jax.experimental.pallas import tpu as pltpu
from jax.experimental.pallas import tpu_sc as plsc

NC = 2
NS = 16
L = 16

N = 10000
E = 160000
NPAD = 10240
RT = NPAD // (NC * NS)


def _sc_mesh():
  return plsc.VectorSubcoreMesh(core_axis_name="c", subcore_axis_name="s")


_SC_PARAMS = pltpu.CompilerParams(needs_layout_passes=False)


_E_PER_W = E // (NC * NS)
_DEG_FULL = _E_PER_W // L
_DEG_TAIL = _E_PER_W - _DEG_FULL * L


def _deg_body(dst_hbm, degp_hbm, dst_v, cnt):
  c = lax.axis_index("c")
  s = lax.axis_index("s")
  wid = c * NS + s
  pltpu.sync_copy(dst_hbm.at[pl.ds(wid * _E_PER_W, _E_PER_W)],
                  dst_v.at[pl.ds(0, _E_PER_W)])

  def zero(i, carry):
    cnt[pl.ds(i * L, L)] = jnp.zeros((L,), jnp.float32)
    return carry
  lax.fori_loop(0, 640, zero, 0)

  ones = jnp.ones((L,), jnp.float32)

  def count(i, carry):
    d = dst_v[pl.ds(i * L, L)]
    plsc.addupdate_scatter(cnt, [d], ones)
    return carry
  lax.fori_loop(0, _DEG_FULL, count, 0)

  if _DEG_TAIL:
    d = dst_v[pl.ds(_DEG_FULL * L, L)]
    msk = lax.iota(jnp.int32, L) < _DEG_TAIL
    d = jnp.where(msk, d, 0)
    plsc.addupdate_scatter(cnt, [d], ones, mask=msk)

  pltpu.sync_copy(cnt, degp_hbm.at[wid])


_deg_kernel = functools.partial(
    pl.kernel,
    out_type=jax.ShapeDtypeStruct((NC * NS, 640 * 16), jnp.float32),
    mesh=_sc_mesh(),
    compiler_params=_SC_PARAMS,
    scratch_types=[
        pltpu.VMEM((_E_PER_W + L,), jnp.int32),
        pltpu.VMEM((640 * 16,), jnp.float32),
    ],
)(_deg_body)


def _make_agg(D, NPASS):
  RP = RT // NPASS
  ECH = 2000
  CG = 64
  GD = D // L

  def body(src_hbm, dst_hbm, h_hbm, out_hbm,
           src_v, dst_v, srcC, dofC, idxg, rows, acc, sem):
    c = lax.axis_index("c")
    s = lax.axis_index("s")
    wid = c * NS + s
    iota = lax.iota(jnp.int32, L)

    def sca(vec, lane):
      return jnp.sum(jnp.where(iota == lane, vec, jnp.int32(0)))

    for p in range(NPASS):
      lo = wid * RT + p * RP

      def zz(i, carry):
        acc[i // GD, pl.ds((i % GD) * L, L)] = jnp.zeros((L,), jnp.float32)
        return carry
      lax.fori_loop(0, (RP + 1) * GD, zz, 0)

      def echunk(ch, carry):
        pltpu.sync_copy(src_hbm.at[pl.ds(ch * ECH, ECH)], src_v)
        pltpu.sync_copy(dst_hbm.at[pl.ds(ch * ECH, ECH)], dst_v)

        def comp(i, base):
          sv = src_v[pl.ds(i * L, L)]
          dv = dst_v[pl.ds(i * L, L)]
          dq = dv - lo
          m = (dq >= 0) & (dq < RP)
          mi = m.astype(jnp.int32)
          pref = plsc.cumsum(mi)
          pos = base + pref - 1
          plsc.store_scatter(srcC, [pos], sv, mask=m)
          plsc.store_scatter(dofC, [pos], dq, mask=m)
          return base + jnp.sum(mi)
        kin = lax.fori_loop(0, ECH // L, comp, jnp.int32(0))

        for t in range(CG // L):
          srcC[pl.ds(kin + t * L, L)] = jnp.zeros((L,), jnp.int32)
          dofC[pl.ds(kin + t * L, L)] = jnp.full((L,), RP, jnp.int32)
        nch = (kin + CG - 1) // CG

        def gchunk(j, carry2):
          for t in range(CG // L):
            idxg[pl.ds(t * L, L)] = srcC[pl.ds(j * CG + t * L, L)]
          pltpu.async_copy(h_hbm.at[idxg], rows, sem).wait()

          def edge(i, carry3):
            dvv = dofC[pl.ds(j * CG + i, L)]
            row = sca(dvv, 0)
            for g in range(GD):
              plsc.addupdate(acc.at[row, pl.ds(g * L, L)],
                             rows[i, pl.ds(g * L, L)])
            return carry3
          lax.fori_loop(0, CG, edge, 0)
          return carry2
        lax.fori_loop(0, nch, gchunk, 0)
        return carry
      lax.fori_loop(0, E // ECH, echunk, 0)

      pltpu.sync_copy(acc.at[pl.ds(0, RP)], out_hbm.at[pl.ds(lo, RP)])

  return pl.kernel(
      body,
      out_type=jax.ShapeDtypeStruct((NPAD, D), jnp.float32),
      mesh=_sc_mesh(),
      compiler_params=_SC_PARAMS,
      scratch_types=[
          pltpu.VMEM((ECH,), jnp.int32),
          pltpu.VMEM((ECH,), jnp.int32),
          pltpu.VMEM((ECH + CG,), jnp.int32),
          pltpu.VMEM((ECH + CG,), jnp.int32),
          pltpu.VMEM((CG,), jnp.int32),
          pltpu.VMEM((CG, D), jnp.float32),
          pltpu.VMEM((RP + 1, D), jnp.float32),
          pltpu.SemaphoreType.DMA,
      ],
  )


MBLK = 400


def _dinv_from_partials(degp_blk):
  deg = jnp.sum(degp_blk, axis=1) + 1.0
  return lax.rsqrt(deg)


def _mm1_body(x_ref, w_ref, degp_ref, o_ref):
  dinv = _dinv_from_partials(degp_ref[...])
  h = jnp.dot(x_ref[...], w_ref[...], preferred_element_type=jnp.float32)
  o_ref[...] = h * dinv[:, None]


def _mm2_body(agg_ref, h_ref, degp_ref, b_ref, w_ref, o_ref):
  dinv = _dinv_from_partials(degp_ref[...])
  z = jnp.maximum(dinv[:, None] * (agg_ref[...] + h_ref[...]) + b_ref[...],
                  0.0)
  h2 = jnp.dot(z, w_ref[...], preferred_element_type=jnp.float32)
  o_ref[...] = h2 * dinv[:, None]


def _final_body(agg_ref, h_ref, degp_ref, b_ref, o_ref):
  dinv = _dinv_from_partials(degp_ref[...])
  o = dinv[:, None] * (agg_ref[...] + h_ref[...]) + b_ref[...]
  o = o - jnp.max(o, axis=1, keepdims=True)
  e = jnp.exp(o)
  o_ref[...] = e / jnp.sum(e, axis=1, keepdims=True)


def _mm1(x, w1, degp):
  d_in, d_hid = w1.shape
  return pl.pallas_call(
      _mm1_body,
      grid=(N // MBLK,),
      in_specs=[
          pl.BlockSpec((MBLK, d_in), lambda i: (i, 0)),
          pl.BlockSpec((d_in, d_hid), lambda i: (0, 0)),
          pl.BlockSpec((MBLK, NC * NS), lambda i: (i, 0)),
      ],
      out_specs=pl.BlockSpec((MBLK, d_hid), lambda i: (i, 0)),
      out_shape=jax.ShapeDtypeStruct((N, d_hid), jnp.float32),
  )(x, w1, degp)


def _mm2(agg1, h1p, degp, b1, w2):
  d_hid, d_out = w2.shape
  return pl.pallas_call(
      _mm2_body,
      grid=(N // MBLK,),
      in_specs=[
          pl.BlockSpec((MBLK, d_hid), lambda i: (i, 0)),
          pl.BlockSpec((MBLK, d_hid), lambda i: (i, 0)),
          pl.BlockSpec((MBLK, NC * NS), lambda i: (i, 0)),
          pl.BlockSpec((1, d_hid), lambda i: (0, 0)),
          pl.BlockSpec((d_hid, d_out), lambda i: (0, 0)),
      ],
      out_specs=pl.BlockSpec((MBLK, d_out), lambda i: (i, 0)),
      out_shape=jax.ShapeDtypeStruct((N, d_out), jnp.float32),
  )(agg1, h1p, degp, b1, w2)


def _final(agg2, h2p, degp, b2):
  d_out = h2p.shape[1]
  return pl.pallas_call(
      _final_body,
      grid=(N // MBLK,),
      in_specs=[
          pl.BlockSpec((MBLK, d_out), lambda i: (i, 0)),
          pl.BlockSpec((MBLK, d_out), lambda i: (i, 0)),
          pl.BlockSpec((MBLK, NC * NS), lambda i: (i, 0)),
          pl.BlockSpec((1, d_out), lambda i: (0, 0)),
      ],
      out_specs=pl.BlockSpec((MBLK, d_out), lambda i: (i, 0)),
      out_shape=jax.ShapeDtypeStruct((N, d_out), jnp.float32),
  )(agg2, h2p, degp, b2)


_agg_512 = _make_agg(512, 2)
_agg_256 = _make_agg(256, 1)


@jax.jit
def kernel(x, edge_index, W1, b1, W2, b2):
  src = edge_index[0]
  dst = edge_index[1]

  degp = _deg_kernel(dst).T[:N]

  h1p = _mm1(x, W1, degp)
  agg1 = _agg_512(src, dst, h1p)[:N]
  h2p = _mm2(agg1, h1p, degp, b1.reshape(1, -1), W2)
  agg2 = _agg_256(src, dst, h2p)[:N]
  return _final(agg2, h2p, degp, b2.reshape(1, -1))


# speedup vs baseline: 1.5070x; 1.5070x over previous
import functools

import jax
import jax.numpy as jnp
from jax import lax
from jax.experimental import pallas as pl
from jax.experimental.pallas import tpu as pltpu
from jax.experimental.pallas import tpu_sc as plsc

NC = 2
NS = 16
L = 16

N = 10000
E = 160000
NPAD = 10240
RT = NPAD // (NC * NS)


def _sc_mesh():
  return plsc.VectorSubcoreMesh(core_axis_name="c", subcore_axis_name="s")


_SC_PARAMS = pltpu.CompilerParams(needs_layout_passes=False)


_E_PER_W = E // (NC * NS)
_DEG_FULL = _E_PER_W // L
_DEG_TAIL = _E_PER_W - _DEG_FULL * L


def _deg_body(dst_hbm, degp_hbm, dst_v, cnt):
  c = lax.axis_index("c")
  s = lax.axis_index("s")
  wid = c * NS + s
  pltpu.sync_copy(dst_hbm.at[pl.ds(wid * _E_PER_W, _E_PER_W)],
                  dst_v.at[pl.ds(0, _E_PER_W)])

  def zero(i, carry):
    cnt[pl.ds(i * L, L)] = jnp.zeros((L,), jnp.float32)
    return carry
  lax.fori_loop(0, 640, zero, 0)

  ones = jnp.ones((L,), jnp.float32)

  def count(i, carry):
    d = dst_v[pl.ds(i * L, L)]
    plsc.addupdate_scatter(cnt, [d], ones)
    return carry
  lax.fori_loop(0, _DEG_FULL, count, 0)

  if _DEG_TAIL:
    d = dst_v[pl.ds(_DEG_FULL * L, L)]
    msk = lax.iota(jnp.int32, L) < _DEG_TAIL
    d = jnp.where(msk, d, 0)
    plsc.addupdate_scatter(cnt, [d], ones, mask=msk)

  pltpu.sync_copy(cnt, degp_hbm.at[wid])


_deg_kernel = functools.partial(
    pl.kernel,
    out_type=jax.ShapeDtypeStruct((NC * NS, 640 * 16), jnp.float32),
    mesh=_sc_mesh(),
    compiler_params=_SC_PARAMS,
    scratch_types=[
        pltpu.VMEM((_E_PER_W + L,), jnp.int32),
        pltpu.VMEM((640 * 16,), jnp.float32),
    ],
)(_deg_body)


def _make_agg(D, NPASS):
  RP = RT // NPASS
  ECH = 2000
  CG = 64
  GD = D // L

  def body(src_hbm, dst_hbm, h_hbm, out_hbm,
           src_v, dst_v, srcC, dofC, idxg, rows, acc, sem):
    c = lax.axis_index("c")
    s = lax.axis_index("s")
    wid = c * NS + s
    iota = lax.iota(jnp.int32, L)

    def sca(vec, lane):
      return jnp.sum(jnp.where(iota == lane, vec, jnp.int32(0)))

    for p in range(NPASS):
      lo = wid * RT + p * RP

      def zz(i, carry):
        acc[i // GD, pl.ds((i % GD) * L, L)] = jnp.zeros((L,), jnp.float32)
        return carry
      lax.fori_loop(0, (RP + 1) * GD, zz, 0)

      def echunk(ch, carry):
        pltpu.sync_copy(src_hbm.at[pl.ds(ch * ECH, ECH)], src_v)
        pltpu.sync_copy(dst_hbm.at[pl.ds(ch * ECH, ECH)], dst_v)

        def comp(i, base):
          sv = src_v[pl.ds(i * L, L)]
          dv = dst_v[pl.ds(i * L, L)]
          dq = dv - lo
          m = (dq >= 0) & (dq < RP)
          mi = m.astype(jnp.int32)
          pref = plsc.cumsum(mi)
          pos = base + pref - 1
          plsc.store_scatter(srcC, [pos], sv, mask=m)
          plsc.store_scatter(dofC, [pos], dq, mask=m)
          return base + jnp.sum(mi)
        kin = lax.fori_loop(0, ECH // L, comp, jnp.int32(0))

        for t in range(CG // L):
          srcC[pl.ds(kin + t * L, L)] = jnp.zeros((L,), jnp.int32)
          dofC[pl.ds(kin + t * L, L)] = jnp.full((L,), RP, jnp.int32)
        nch = (kin + CG - 1) // CG

        def gchunk(j, carry2):
          for t in range(CG // L):
            idxg[pl.ds(t * L, L)] = srcC[pl.ds(j * CG + t * L, L)]
          pltpu.async_copy(h_hbm.at[idxg], rows, sem).wait()

          def edge(i, carry3):
            dvv = dofC[pl.ds(j * CG + i, L)]
            row = sca(dvv, 0)
            for g in range(GD):
              plsc.addupdate(acc.at[row, pl.ds(g * L, L)],
                             rows[i, pl.ds(g * L, L)])
            return carry3
          lax.fori_loop(0, CG, edge, 0)
          return carry2
        lax.fori_loop(0, nch, gchunk, 0)
        return carry
      lax.fori_loop(0, E // ECH, echunk, 0)

      pltpu.sync_copy(acc.at[pl.ds(0, RP)], out_hbm.at[pl.ds(lo, RP)])

  return pl.kernel(
      body,
      out_type=jax.ShapeDtypeStruct((NPAD, D), jnp.float32),
      mesh=_sc_mesh(),
      compiler_params=_SC_PARAMS,
      scratch_types=[
          pltpu.VMEM((ECH,), jnp.int32),
          pltpu.VMEM((ECH,), jnp.int32),
          pltpu.VMEM((ECH + CG,), jnp.int32),
          pltpu.VMEM((ECH + CG,), jnp.int32),
          pltpu.VMEM((CG,), jnp.int32),
          pltpu.VMEM((CG, D), jnp.float32),
          pltpu.VMEM((RP + 1, D), jnp.float32),
          pltpu.SemaphoreType.DMA,
      ],
  )


MBLK = 400


def _dinv_from_partials(degp_blk):
  deg = jnp.sum(degp_blk, axis=1) + 1.0
  return lax.rsqrt(deg)


def _scale_body(x_ref, degp_ref, o_ref):
  dinv = _dinv_from_partials(degp_ref[...])
  o_ref[...] = x_ref[...] * dinv[:, None]


def _mm12_body(agg_ref, y_ref, degp_ref, b_ref, w1_ref, w2_ref, o_ref):
  dinv = _dinv_from_partials(degp_ref[...])
  h1 = jnp.dot(agg_ref[...] + y_ref[...], w1_ref[...],
               preferred_element_type=jnp.float32)
  z = jnp.maximum(dinv[:, None] * h1 + b_ref[...], 0.0)
  h2 = jnp.dot(z, w2_ref[...], preferred_element_type=jnp.float32)
  o_ref[...] = h2 * dinv[:, None]


def _final_body(agg_ref, h_ref, degp_ref, b_ref, o_ref):
  dinv = _dinv_from_partials(degp_ref[...])
  o = dinv[:, None] * (agg_ref[...] + h_ref[...]) + b_ref[...]
  o = o - jnp.max(o, axis=1, keepdims=True)
  e = jnp.exp(o)
  o_ref[...] = e / jnp.sum(e, axis=1, keepdims=True)


def _scale(x, degp):
  d_in = x.shape[1]
  return pl.pallas_call(
      _scale_body,
      grid=(N // MBLK,),
      in_specs=[
          pl.BlockSpec((MBLK, d_in), lambda i: (i, 0)),
          pl.BlockSpec((MBLK, NC * NS), lambda i: (i, 0)),
      ],
      out_specs=pl.BlockSpec((MBLK, d_in), lambda i: (i, 0)),
      out_shape=jax.ShapeDtypeStruct((N, d_in), jnp.float32),
  )(x, degp)


def _mm12(aggx, y, degp, b1, w1, w2):
  d_in, d_hid = w1.shape
  d_out = w2.shape[1]
  return pl.pallas_call(
      _mm12_body,
      grid=(N // MBLK,),
      in_specs=[
          pl.BlockSpec((MBLK, d_in), lambda i: (i, 0)),
          pl.BlockSpec((MBLK, d_in), lambda i: (i, 0)),
          pl.BlockSpec((MBLK, NC * NS), lambda i: (i, 0)),
          pl.BlockSpec((1, d_hid), lambda i: (0, 0)),
          pl.BlockSpec((d_in, d_hid), lambda i: (0, 0)),
          pl.BlockSpec((d_hid, d_out), lambda i: (0, 0)),
      ],
      out_specs=pl.BlockSpec((MBLK, d_out), lambda i: (i, 0)),
      out_shape=jax.ShapeDtypeStruct((N, d_out), jnp.float32),
  )(aggx, y, degp, b1, w1, w2)


def _final(agg2, h2p, degp, b2):
  d_out = h2p.shape[1]
  return pl.pallas_call(
      _final_body,
      grid=(N // MBLK,),
      in_specs=[
          pl.BlockSpec((MBLK, d_out), lambda i: (i, 0)),
          pl.BlockSpec((MBLK, d_out), lambda i: (i, 0)),
          pl.BlockSpec((MBLK, NC * NS), lambda i: (i, 0)),
          pl.BlockSpec((1, d_out), lambda i: (0, 0)),
      ],
      out_specs=pl.BlockSpec((MBLK, d_out), lambda i: (i, 0)),
      out_shape=jax.ShapeDtypeStruct((N, d_out), jnp.float32),
  )(agg2, h2p, degp, b2)


_agg_256 = _make_agg(256, 1)


@jax.jit
def kernel(x, edge_index, W1, b1, W2, b2):
  src = edge_index[0]
  dst = edge_index[1]

  degp = _deg_kernel(dst).T[:N]

  y = _scale(x, degp)
  aggx = _agg_256(src, dst, y)[:N]
  h2p = _mm12(aggx, y, degp, b1.reshape(1, -1), W1, W2)
  agg2 = _agg_256(src, dst, h2p)[:N]
  return _final(agg2, h2p, degp, b2.reshape(1, -1))


# speedup vs baseline: 2.8328x; 1.8798x over previous
import functools

import jax
import jax.numpy as jnp
from jax import lax
from jax.experimental import pallas as pl
from jax.experimental.pallas import tpu as pltpu
from jax.experimental.pallas import tpu_sc as plsc

NC = 2
NS = 16
L = 16

N = 10000
E = 160000
NPAD = 10240
RT = NPAD // (NC * NS)


def _sc_mesh():
  return plsc.VectorSubcoreMesh(core_axis_name="c", subcore_axis_name="s")


_SC_PARAMS = pltpu.CompilerParams(needs_layout_passes=False)


_E_PER_W = E // (NC * NS)
_DEG_FULL = _E_PER_W // L
_DEG_TAIL = _E_PER_W - _DEG_FULL * L


def _deg_body(dst_hbm, degp_hbm, dst_v, cnt):
  c = lax.axis_index("c")
  s = lax.axis_index("s")
  wid = c * NS + s
  pltpu.sync_copy(dst_hbm.at[pl.ds(wid * _E_PER_W, _E_PER_W)],
                  dst_v.at[pl.ds(0, _E_PER_W)])

  def zero(i, carry):
    cnt[pl.ds(i * L, L)] = jnp.zeros((L,), jnp.float32)
    return carry
  lax.fori_loop(0, 640, zero, 0)

  ones = jnp.ones((L,), jnp.float32)

  def count(i, carry):
    d = dst_v[pl.ds(i * L, L)]
    plsc.addupdate_scatter(cnt, [d], ones)
    return carry
  lax.fori_loop(0, _DEG_FULL, count, 0)

  if _DEG_TAIL:
    d = dst_v[pl.ds(_DEG_FULL * L, L)]
    msk = lax.iota(jnp.int32, L) < _DEG_TAIL
    d = jnp.where(msk, d, 0)
    plsc.addupdate_scatter(cnt, [d], ones, mask=msk)

  pltpu.sync_copy(cnt, degp_hbm.at[wid])


_deg_kernel = functools.partial(
    pl.kernel,
    out_type=jax.ShapeDtypeStruct((NC * NS, 640 * 16), jnp.float32),
    mesh=_sc_mesh(),
    compiler_params=_SC_PARAMS,
    scratch_types=[
        pltpu.VMEM((_E_PER_W + L,), jnp.int32),
        pltpu.VMEM((640 * 16,), jnp.float32),
    ],
)(_deg_body)


def _make_agg(D, NPASS):
  RP = RT // NPASS
  ECH = 4000
  CG = 64
  GD = D // L

  def body(src_hbm, dst_hbm, h_hbm, out_hbm,
           src_v, dst_v, srcC, dofC, idxg, rows, acc, sem):
    c = lax.axis_index("c")
    s = lax.axis_index("s")
    wid = c * NS + s
    iota = lax.iota(jnp.int32, L)

    def sca(vec, lane):
      return jnp.sum(jnp.where(iota == lane, vec, jnp.int32(0)))

    for p in range(NPASS):
      lo = wid * RT + p * RP

      def zz(i, carry):
        acc[i // GD, pl.ds((i % GD) * L, L)] = jnp.zeros((L,), jnp.float32)
        return carry
      lax.fori_loop(0, (RP + 1) * GD, zz, 0)

      def echunk(ch, carry):
        pltpu.sync_copy(src_hbm.at[pl.ds(ch * ECH, ECH)], src_v)
        pltpu.sync_copy(dst_hbm.at[pl.ds(ch * ECH, ECH)], dst_v)

        def comp(i, base):
          sv = src_v[pl.ds(i * L, L)]
          dv = dst_v[pl.ds(i * L, L)]
          dq = dv - lo
          m = (dq >= 0) & (dq < RP)
          mi = m.astype(jnp.int32)
          pref = plsc.cumsum(mi)
          pos = base + pref - 1
          plsc.store_scatter(srcC, [pos], sv, mask=m)
          plsc.store_scatter(dofC, [pos], dq, mask=m)
          return base + jnp.sum(mi)
        kin = lax.fori_loop(0, ECH // L, comp, jnp.int32(0))

        for t in range(CG // L):
          srcC[pl.ds(kin + t * L, L)] = jnp.zeros((L,), jnp.int32)
          dofC[pl.ds(kin + t * L, L)] = jnp.full((L,), RP, jnp.int32)
        nch = (kin + CG - 1) // CG

        def gchunk(j, carry2):
          for t in range(CG // L):
            idxg[pl.ds(t * L, L)] = srcC[pl.ds(j * CG + t * L, L)]
          pltpu.async_copy(h_hbm.at[idxg], rows, sem).wait()

          def edge(i, carry3):
            dvv = dofC[pl.ds(j * CG + i, L)]
            row = sca(dvv, 0)
            for g in range(GD):
              plsc.addupdate(acc.at[row, pl.ds(g * L, L)],
                             rows[i, pl.ds(g * L, L)])
            return carry3
          lax.fori_loop(0, CG, edge, 0)
          return carry2
        lax.fori_loop(0, nch, gchunk, 0)
        return carry
      lax.fori_loop(0, E // ECH, echunk, 0)

      pltpu.sync_copy(acc.at[pl.ds(0, RP)], out_hbm.at[pl.ds(lo, RP)])

  return pl.kernel(
      body,
      out_type=jax.ShapeDtypeStruct((NPAD, D), jnp.float32),
      mesh=_sc_mesh(),
      compiler_params=_SC_PARAMS,
      scratch_types=[
          pltpu.VMEM((ECH,), jnp.int32),
          pltpu.VMEM((ECH,), jnp.int32),
          pltpu.VMEM((ECH + CG,), jnp.int32),
          pltpu.VMEM((ECH + CG,), jnp.int32),
          pltpu.VMEM((CG,), jnp.int32),
          pltpu.VMEM((CG, D), jnp.float32),
          pltpu.VMEM((RP + 1, D), jnp.float32),
          pltpu.SemaphoreType.DMA,
      ],
  )


MBLK = 400


def _dinv_from_partials(degp_blk):
  deg = jnp.sum(degp_blk, axis=1) + 1.0
  return lax.rsqrt(deg)


def _scale_body(x_ref, degp_ref, o_ref):
  dinv = _dinv_from_partials(degp_ref[...])
  o_ref[...] = x_ref[...] * dinv[:, None]


def _mm12_body(agg_ref, y_ref, degp_ref, b_ref, w1_ref, w2_ref, o_ref):
  dinv = _dinv_from_partials(degp_ref[...])
  h1 = jnp.dot(agg_ref[...] + y_ref[...], w1_ref[...],
               preferred_element_type=jnp.float32)
  z = jnp.maximum(dinv[:, None] * h1 + b_ref[...], 0.0)
  h2 = jnp.dot(z, w2_ref[...], preferred_element_type=jnp.float32)
  o_ref[...] = h2 * dinv[:, None]


def _final_body(agg_ref, h_ref, degp_ref, b_ref, o_ref):
  dinv = _dinv_from_partials(degp_ref[...])
  o = dinv[:, None] * (agg_ref[...] + h_ref[...]) + b_ref[...]
  o = o - jnp.max(o, axis=1, keepdims=True)
  e = jnp.exp(o)
  o_ref[...] = e / jnp.sum(e, axis=1, keepdims=True)


def _scale(x, degp):
  d_in = x.shape[1]
  return pl.pallas_call(
      _scale_body,
      grid=(N // MBLK,),
      in_specs=[
          pl.BlockSpec((MBLK, d_in), lambda i: (i, 0)),
          pl.BlockSpec((MBLK, NC * NS), lambda i: (i, 0)),
      ],
      out_specs=pl.BlockSpec((MBLK, d_in), lambda i: (i, 0)),
      out_shape=jax.ShapeDtypeStruct((N, d_in), jnp.float32),
  )(x, degp)


def _mm12(aggx, y, degp, b1, w1, w2):
  d_in, d_hid = w1.shape
  d_out = w2.shape[1]
  return pl.pallas_call(
      _mm12_body,
      grid=(N // MBLK,),
      in_specs=[
          pl.BlockSpec((MBLK, d_in), lambda i: (i, 0)),
          pl.BlockSpec((MBLK, d_in), lambda i: (i, 0)),
          pl.BlockSpec((MBLK, NC * NS), lambda i: (i, 0)),
          pl.BlockSpec((1, d_hid), lambda i: (0, 0)),
          pl.BlockSpec((d_in, d_hid), lambda i: (0, 0)),
          pl.BlockSpec((d_hid, d_out), lambda i: (0, 0)),
      ],
      out_specs=pl.BlockSpec((MBLK, d_out), lambda i: (i, 0)),
      out_shape=jax.ShapeDtypeStruct((N, d_out), jnp.float32),
  )(aggx, y, degp, b1, w1, w2)


def _final(agg2, h2p, degp, b2):
  d_out = h2p.shape[1]
  return pl.pallas_call(
      _final_body,
      grid=(N // MBLK,),
      in_specs=[
          pl.BlockSpec((MBLK, d_out), lambda i: (i, 0)),
          pl.BlockSpec((MBLK, d_out), lambda i: (i, 0)),
          pl.BlockSpec((MBLK, NC * NS), lambda i: (i, 0)),
          pl.BlockSpec((1, d_out), lambda i: (0, 0)),
      ],
      out_specs=pl.BlockSpec((MBLK, d_out), lambda i: (i, 0)),
      out_shape=jax.ShapeDtypeStruct((N, d_out), jnp.float32),
  )(agg2, h2p, degp, b2)


_agg_256 = _make_agg(256, 1)


@jax.jit
def kernel(x, edge_index, W1, b1, W2, b2):
  src = edge_index[0]
  dst = edge_index[1]

  degp = _deg_kernel(dst).T[:N]

  y = _scale(x, degp)
  aggx = _agg_256(src, dst, y)[:N]
  h2p = _mm12(aggx, y, degp, b1.reshape(1, -1), W1, W2)
  agg2 = _agg_256(src, dst, h2p)[:N]
  return _final(agg2, h2p, degp, b2.reshape(1, -1))


# speedup vs baseline: 2.8398x; 1.0025x over previous
import functools

import jax
import jax.numpy as jnp
from jax import lax
from jax.experimental import pallas as pl
from jax.experimental.pallas import tpu as pltpu
from jax.experimental.pallas import tpu_sc as plsc

NC = 2
NS = 16
L = 16

N = 10000
E = 160000
NPAD = 10240
RT = NPAD // (NC * NS)


def _sc_mesh():
  return plsc.VectorSubcoreMesh(core_axis_name="c", subcore_axis_name="s")


_SC_PARAMS = pltpu.CompilerParams(needs_layout_passes=False)


_E_PER_W = E // (NC * NS)
_DEG_FULL = _E_PER_W // L
_DEG_TAIL = _E_PER_W - _DEG_FULL * L


def _deg_body(dst_hbm, degp_hbm, dst_v, cnt):
  c = lax.axis_index("c")
  s = lax.axis_index("s")
  wid = c * NS + s
  pltpu.sync_copy(dst_hbm.at[pl.ds(wid * _E_PER_W, _E_PER_W)],
                  dst_v.at[pl.ds(0, _E_PER_W)])

  def zero(i, carry):
    cnt[pl.ds(i * L, L)] = jnp.zeros((L,), jnp.float32)
    return carry
  lax.fori_loop(0, 640, zero, 0)

  ones = jnp.ones((L,), jnp.float32)

  def count(i, carry):
    d = dst_v[pl.ds(i * L, L)]
    plsc.addupdate_scatter(cnt, [d], ones)
    return carry
  lax.fori_loop(0, _DEG_FULL, count, 0)

  if _DEG_TAIL:
    d = dst_v[pl.ds(_DEG_FULL * L, L)]
    msk = lax.iota(jnp.int32, L) < _DEG_TAIL
    d = jnp.where(msk, d, 0)
    plsc.addupdate_scatter(cnt, [d], ones, mask=msk)

  pltpu.sync_copy(cnt, degp_hbm.at[wid])


_deg_kernel = functools.partial(
    pl.kernel,
    out_type=jax.ShapeDtypeStruct((NC * NS, 640 * 16), jnp.float32),
    mesh=_sc_mesh(),
    compiler_params=_SC_PARAMS,
    scratch_types=[
        pltpu.VMEM((_E_PER_W + L,), jnp.int32),
        pltpu.VMEM((640 * 16,), jnp.float32),
    ],
)(_deg_body)


def _make_agg(D, NPASS):
  RP = RT // NPASS
  ECH = 4000
  CG = 64
  GD = D // L

  NCH = E // ECH

  def body(src_hbm, dst_hbm, h_hbm, out_hbm,
           srcA, dstA, srcB, dstB, srcC, dofC, idxg, rows, acc,
           semSA, semDA, semSB, semDB, sem):
    c = lax.axis_index("c")
    s = lax.axis_index("s")
    wid = c * NS + s
    iota = lax.iota(jnp.int32, L)

    def sca(vec, lane):
      return jnp.sum(jnp.where(iota == lane, vec, jnp.int32(0)))

    def start_edge(ch, sbuf, dbuf, ssem, dsem):
      pltpu.async_copy(src_hbm.at[pl.ds(ch * ECH, ECH)], sbuf, ssem)
      pltpu.async_copy(dst_hbm.at[pl.ds(ch * ECH, ECH)], dbuf, dsem)

    def wait_edge(ch, sbuf, dbuf, ssem, dsem):
      pltpu.make_async_copy(src_hbm.at[pl.ds(ch * ECH, ECH)], sbuf,
                            ssem).wait()
      pltpu.make_async_copy(dst_hbm.at[pl.ds(ch * ECH, ECH)], dbuf,
                            dsem).wait()

    for p in range(NPASS):
      lo = wid * RT + p * RP

      def zz(i, carry):
        acc[i // GD, pl.ds((i % GD) * L, L)] = jnp.zeros((L,), jnp.float32)
        return carry
      lax.fori_loop(0, (RP + 1) * GD, zz, 0)

      def process(src_v, dst_v):
        def comp(i, base):
          sv = src_v[pl.ds(i * L, L)]
          dv = dst_v[pl.ds(i * L, L)]
          dq = dv - lo
          m = (dq >= 0) & (dq < RP)
          mi = m.astype(jnp.int32)
          pref = plsc.cumsum(mi)
          pos = base + pref - 1
          plsc.store_scatter(srcC, [pos], sv, mask=m)
          plsc.store_scatter(dofC, [pos], dq, mask=m)
          return base + jnp.sum(mi)
        kin = lax.fori_loop(0, ECH // L, comp, jnp.int32(0))

        for t in range(CG // L):
          srcC[pl.ds(kin + t * L, L)] = jnp.zeros((L,), jnp.int32)
          dofC[pl.ds(kin + t * L, L)] = jnp.full((L,), RP, jnp.int32)
        nch = (kin + CG - 1) // CG

        def gchunk(j, carry2):
          for t in range(CG // L):
            idxg[pl.ds(t * L, L)] = srcC[pl.ds(j * CG + t * L, L)]
          pltpu.async_copy(h_hbm.at[idxg], rows, sem).wait()

          def edge(i, carry3):
            dvv = dofC[pl.ds(j * CG + i, L)]
            row = sca(dvv, 0)
            for g in range(GD):
              plsc.addupdate(acc.at[row, pl.ds(g * L, L)],
                             rows[i, pl.ds(g * L, L)])
            return carry3
          lax.fori_loop(0, CG, edge, 0)
          return carry2
        lax.fori_loop(0, nch, gchunk, 0)

      start_edge(0, srcA, dstA, semSA, semDA)

      def pair(k, carry):
        c0 = 2 * k
        start_edge(c0 + 1, srcB, dstB, semSB, semDB)
        wait_edge(c0, srcA, dstA, semSA, semDA)
        process(srcA, dstA)
        start_edge(lax.rem(c0 + 2, NCH), srcA, dstA, semSA, semDA)
        wait_edge(c0 + 1, srcB, dstB, semSB, semDB)
        process(srcB, dstB)
        return carry
      lax.fori_loop(0, NCH // 2, pair, 0)

      wait_edge(0, srcA, dstA, semSA, semDA)

      pltpu.sync_copy(acc.at[pl.ds(0, RP)], out_hbm.at[pl.ds(lo, RP)])

  return pl.kernel(
      body,
      out_type=jax.ShapeDtypeStruct((NPAD, D), jnp.float32),
      mesh=_sc_mesh(),
      compiler_params=_SC_PARAMS,
      scratch_types=[
          pltpu.VMEM((ECH,), jnp.int32),
          pltpu.VMEM((ECH,), jnp.int32),
          pltpu.VMEM((ECH,), jnp.int32),
          pltpu.VMEM((ECH,), jnp.int32),
          pltpu.VMEM((ECH + CG,), jnp.int32),
          pltpu.VMEM((ECH + CG,), jnp.int32),
          pltpu.VMEM((CG,), jnp.int32),
          pltpu.VMEM((CG, D), jnp.float32),
          pltpu.VMEM((RP + 1, D), jnp.float32),
          pltpu.SemaphoreType.DMA,
          pltpu.SemaphoreType.DMA,
          pltpu.SemaphoreType.DMA,
          pltpu.SemaphoreType.DMA,
          pltpu.SemaphoreType.DMA,
      ],
  )


MBLK = 400


def _dinv_from_partials(degp_blk):
  deg = jnp.sum(degp_blk, axis=1) + 1.0
  return lax.rsqrt(deg)


def _scale_body(x_ref, degp_ref, o_ref):
  dinv = _dinv_from_partials(degp_ref[...])
  o_ref[...] = x_ref[...] * dinv[:, None]


def _mm12_body(agg_ref, y_ref, degp_ref, b_ref, w1_ref, w2_ref, o_ref):
  dinv = _dinv_from_partials(degp_ref[...])
  h1 = jnp.dot(agg_ref[...] + y_ref[...], w1_ref[...],
               preferred_element_type=jnp.float32)
  z = jnp.maximum(dinv[:, None] * h1 + b_ref[...], 0.0)
  h2 = jnp.dot(z, w2_ref[...], preferred_element_type=jnp.float32)
  o_ref[...] = h2 * dinv[:, None]


def _final_body(agg_ref, h_ref, degp_ref, b_ref, o_ref):
  dinv = _dinv_from_partials(degp_ref[...])
  o = dinv[:, None] * (agg_ref[...] + h_ref[...]) + b_ref[...]
  o = o - jnp.max(o, axis=1, keepdims=True)
  e = jnp.exp(o)
  o_ref[...] = e / jnp.sum(e, axis=1, keepdims=True)


def _scale(x, degp):
  d_in = x.shape[1]
  return pl.pallas_call(
      _scale_body,
      grid=(N // MBLK,),
      in_specs=[
          pl.BlockSpec((MBLK, d_in), lambda i: (i, 0)),
          pl.BlockSpec((MBLK, NC * NS), lambda i: (i, 0)),
      ],
      out_specs=pl.BlockSpec((MBLK, d_in), lambda i: (i, 0)),
      out_shape=jax.ShapeDtypeStruct((N, d_in), jnp.float32),
  )(x, degp)


def _mm12(aggx, y, degp, b1, w1, w2):
  d_in, d_hid = w1.shape
  d_out = w2.shape[1]
  return pl.pallas_call(
      _mm12_body,
      grid=(N // MBLK,),
      in_specs=[
          pl.BlockSpec((MBLK, d_in), lambda i: (i, 0)),
          pl.BlockSpec((MBLK, d_in), lambda i: (i, 0)),
          pl.BlockSpec((MBLK, NC * NS), lambda i: (i, 0)),
          pl.BlockSpec((1, d_hid), lambda i: (0, 0)),
          pl.BlockSpec((d_in, d_hid), lambda i: (0, 0)),
          pl.BlockSpec((d_hid, d_out), lambda i: (0, 0)),
      ],
      out_specs=pl.BlockSpec((MBLK, d_out), lambda i: (i, 0)),
      out_shape=jax.ShapeDtypeStruct((N, d_out), jnp.float32),
  )(aggx, y, degp, b1, w1, w2)


def _final(agg2, h2p, degp, b2):
  d_out = h2p.shape[1]
  return pl.pallas_call(
      _final_body,
      grid=(N // MBLK,),
      in_specs=[
          pl.BlockSpec((MBLK, d_out), lambda i: (i, 0)),
          pl.BlockSpec((MBLK, d_out), lambda i: (i, 0)),
          pl.BlockSpec((MBLK, NC * NS), lambda i: (i, 0)),
          pl.BlockSpec((1, d_out), lambda i: (0, 0)),
      ],
      out_specs=pl.BlockSpec((MBLK, d_out), lambda i: (i, 0)),
      out_shape=jax.ShapeDtypeStruct((N, d_out), jnp.float32),
  )(agg2, h2p, degp, b2)


_agg_256 = _make_agg(256, 1)


@jax.jit
def kernel(x, edge_index, W1, b1, W2, b2):
  src = edge_index[0]
  dst = edge_index[1]

  degp = _deg_kernel(dst).T[:N]

  y = _scale(x, degp)
  aggx = _agg_256(src, dst, y)[:N]
  h2p = _mm12(aggx, y, degp, b1.reshape(1, -1), W1, W2)
  agg2 = _agg_256(src, dst, h2p)[:N]
  return _final(agg2, h2p, degp, b2.reshape(1, -1))


# speedup vs baseline: 2.9132x; 1.0258x over previous
import functools

import jax
import jax.numpy as jnp
from jax import lax
from jax.experimental import pallas as pl
from jax.experimental.pallas import tpu as pltpu
from jax.experimental.pallas import tpu_sc as plsc

NC = 2
NS = 16
L = 16

N = 10000
E = 160000
NPAD = 10240
RT = NPAD // (NC * NS)


def _sc_mesh():
  return plsc.VectorSubcoreMesh(core_axis_name="c", subcore_axis_name="s")


_SC_PARAMS = pltpu.CompilerParams(needs_layout_passes=False)


_E_PER_W = E // (NC * NS)
_DEG_FULL = _E_PER_W // L
_DEG_TAIL = _E_PER_W - _DEG_FULL * L


def _deg_body(dst_hbm, degp_hbm, dst_v, cnt):
  c = lax.axis_index("c")
  s = lax.axis_index("s")
  wid = c * NS + s
  pltpu.sync_copy(dst_hbm.at[pl.ds(wid * _E_PER_W, _E_PER_W)],
                  dst_v.at[pl.ds(0, _E_PER_W)])

  def zero(i, carry):
    cnt[pl.ds(i * L, L)] = jnp.zeros((L,), jnp.float32)
    return carry
  lax.fori_loop(0, 640, zero, 0)

  ones = jnp.ones((L,), jnp.float32)

  def count(i, carry):
    d = dst_v[pl.ds(i * L, L)]
    plsc.addupdate_scatter(cnt, [d], ones)
    return carry
  lax.fori_loop(0, _DEG_FULL, count, 0)

  if _DEG_TAIL:
    d = dst_v[pl.ds(_DEG_FULL * L, L)]
    msk = lax.iota(jnp.int32, L) < _DEG_TAIL
    d = jnp.where(msk, d, 0)
    plsc.addupdate_scatter(cnt, [d], ones, mask=msk)

  pltpu.sync_copy(cnt, degp_hbm.at[wid])


_deg_kernel = functools.partial(
    pl.kernel,
    out_type=jax.ShapeDtypeStruct((NC * NS, 640 * 16), jnp.float32),
    mesh=_sc_mesh(),
    compiler_params=_SC_PARAMS,
    scratch_types=[
        pltpu.VMEM((_E_PER_W + L,), jnp.int32),
        pltpu.VMEM((640 * 16,), jnp.float32),
    ],
)(_deg_body)


def _make_agg(D, NPASS):
  RP = RT // NPASS
  ECH = 4000
  CG = 64
  GD = D // L
  DP = D // 2
  GP = DP // L

  NCH = E // ECH

  def body(src_hbm, dst_hbm, h_hbm, out_hbm,
           srcA, dstA, srcB, dstB, srcC, dofC, idxg, rows, acc,
           semSA, semDA, semSB, semDB, sem):
    c = lax.axis_index("c")
    s = lax.axis_index("s")
    wid = c * NS + s
    iota = lax.iota(jnp.int32, L)

    def sca(vec, lane):
      return jnp.sum(jnp.where(iota == lane, vec, jnp.int32(0)))

    def start_edge(ch, sbuf, dbuf, ssem, dsem):
      pltpu.async_copy(src_hbm.at[pl.ds(ch * ECH, ECH)], sbuf, ssem)
      pltpu.async_copy(dst_hbm.at[pl.ds(ch * ECH, ECH)], dbuf, dsem)

    def wait_edge(ch, sbuf, dbuf, ssem, dsem):
      pltpu.make_async_copy(src_hbm.at[pl.ds(ch * ECH, ECH)], sbuf,
                            ssem).wait()
      pltpu.make_async_copy(dst_hbm.at[pl.ds(ch * ECH, ECH)], dbuf,
                            dsem).wait()

    for p in range(NPASS):
      lo = wid * RT + p * RP

      def zz(i, carry):
        acc[i // GD, pl.ds((i % GD) * L, L)] = jnp.zeros((L,), jnp.float32)
        return carry
      lax.fori_loop(0, (RP + 1) * GD, zz, 0)

      def process(src_v, dst_v):
        def comp(i, base):
          sv = src_v[pl.ds(i * L, L)]
          dv = dst_v[pl.ds(i * L, L)]
          dq = dv - lo
          m = (dq >= 0) & (dq < RP)
          mi = m.astype(jnp.int32)
          pref = plsc.cumsum(mi)
          pos = base + pref - 1
          plsc.store_scatter(srcC, [pos], sv, mask=m)
          plsc.store_scatter(dofC, [pos], dq, mask=m)
          return base + jnp.sum(mi)
        kin = lax.fori_loop(0, ECH // L, comp, jnp.int32(0))

        for t in range(CG // L):
          srcC[pl.ds(kin + t * L, L)] = jnp.zeros((L,), jnp.int32)
          dofC[pl.ds(kin + t * L, L)] = jnp.full((L,), RP, jnp.int32)
        nch = (kin + CG - 1) // CG

        def gchunk(j, carry2):
          for t in range(CG // L):
            idxg[pl.ds(t * L, L)] = srcC[pl.ds(j * CG + t * L, L)]
          pltpu.async_copy(h_hbm.at[idxg], rows, sem).wait()

          def edge(i, carry3):
            dvv = dofC[pl.ds(j * CG + i, L)]
            row = sca(dvv, 0)
            for g in range(GP):
              v = rows[i, pl.ds(g * L, L)]
              lo = lax.bitcast_convert_type(v << 16, jnp.float32)
              hi = lax.bitcast_convert_type(
                  jnp.bitwise_and(v, jnp.int32(-65536)), jnp.float32)
              plsc.addupdate(acc.at[row, pl.ds(g * L, L)], lo)
              plsc.addupdate(acc.at[row, pl.ds(DP + g * L, L)], hi)
            return carry3
          lax.fori_loop(0, CG, edge, 0)
          return carry2
        lax.fori_loop(0, nch, gchunk, 0)

      start_edge(0, srcA, dstA, semSA, semDA)

      def pair(k, carry):
        c0 = 2 * k
        start_edge(c0 + 1, srcB, dstB, semSB, semDB)
        wait_edge(c0, srcA, dstA, semSA, semDA)
        process(srcA, dstA)
        start_edge(lax.rem(c0 + 2, NCH), srcA, dstA, semSA, semDA)
        wait_edge(c0 + 1, srcB, dstB, semSB, semDB)
        process(srcB, dstB)
        return carry
      lax.fori_loop(0, NCH // 2, pair, 0)

      wait_edge(0, srcA, dstA, semSA, semDA)

      pltpu.sync_copy(acc.at[pl.ds(0, RP)], out_hbm.at[pl.ds(lo, RP)])

  return pl.kernel(
      body,
      out_type=jax.ShapeDtypeStruct((NPAD, D), jnp.float32),
      mesh=_sc_mesh(),
      compiler_params=_SC_PARAMS,
      scratch_types=[
          pltpu.VMEM((ECH,), jnp.int32),
          pltpu.VMEM((ECH,), jnp.int32),
          pltpu.VMEM((ECH,), jnp.int32),
          pltpu.VMEM((ECH,), jnp.int32),
          pltpu.VMEM((ECH + CG,), jnp.int32),
          pltpu.VMEM((ECH + CG,), jnp.int32),
          pltpu.VMEM((CG,), jnp.int32),
          pltpu.VMEM((CG, DP), jnp.int32),
          pltpu.VMEM((RP + 1, D), jnp.float32),
          pltpu.SemaphoreType.DMA,
          pltpu.SemaphoreType.DMA,
          pltpu.SemaphoreType.DMA,
          pltpu.SemaphoreType.DMA,
          pltpu.SemaphoreType.DMA,
      ],
  )


MBLK = 400


def _dinv_from_partials(degp_blk):
  deg = jnp.sum(degp_blk, axis=1) + 1.0
  return lax.rsqrt(deg)


def _pack_rows(a):
  h = a.shape[1] // 2
  a16 = a.astype(jnp.bfloat16)
  lo = lax.bitcast_convert_type(a16[:, :h], jnp.uint16).astype(jnp.uint32)
  hi = lax.bitcast_convert_type(a16[:, h:], jnp.uint16).astype(jnp.uint32)
  return lax.bitcast_convert_type(lo | (hi << 16), jnp.int32)


def _scale_body(x_ref, degp_ref, o_ref, p_ref):
  dinv = _dinv_from_partials(degp_ref[...])
  y = x_ref[...] * dinv[:, None]
  o_ref[...] = y
  p_ref[...] = _pack_rows(y)


def _mm12_body(agg_ref, y_ref, degp_ref, b_ref, w1_ref, w2_ref, o_ref,
               p_ref):
  dinv = _dinv_from_partials(degp_ref[...])
  h1 = jnp.dot(agg_ref[...] + y_ref[...], w1_ref[...],
               preferred_element_type=jnp.float32)
  z = jnp.maximum(dinv[:, None] * h1 + b_ref[...], 0.0)
  h2 = jnp.dot(z, w2_ref[...], preferred_element_type=jnp.float32)
  h2 = h2 * dinv[:, None]
  o_ref[...] = h2
  p_ref[...] = _pack_rows(h2)


def _final_body(agg_ref, h_ref, degp_ref, b_ref, o_ref):
  dinv = _dinv_from_partials(degp_ref[...])
  o = dinv[:, None] * (agg_ref[...] + h_ref[...]) + b_ref[...]
  o = o - jnp.max(o, axis=1, keepdims=True)
  e = jnp.exp(o)
  o_ref[...] = e / jnp.sum(e, axis=1, keepdims=True)


def _scale(x, degp):
  d_in = x.shape[1]
  return pl.pallas_call(
      _scale_body,
      grid=(N // MBLK,),
      in_specs=[
          pl.BlockSpec((MBLK, d_in), lambda i: (i, 0)),
          pl.BlockSpec((MBLK, NC * NS), lambda i: (i, 0)),
      ],
      out_specs=[
          pl.BlockSpec((MBLK, d_in), lambda i: (i, 0)),
          pl.BlockSpec((MBLK, d_in // 2), lambda i: (i, 0)),
      ],
      out_shape=[
          jax.ShapeDtypeStruct((N, d_in), jnp.float32),
          jax.ShapeDtypeStruct((N, d_in // 2), jnp.int32),
      ],
  )(x, degp)


def _mm12(aggx, y, degp, b1, w1, w2):
  d_in, d_hid = w1.shape
  d_out = w2.shape[1]
  return pl.pallas_call(
      _mm12_body,
      grid=(N // MBLK,),
      in_specs=[
          pl.BlockSpec((MBLK, d_in), lambda i: (i, 0)),
          pl.BlockSpec((MBLK, d_in), lambda i: (i, 0)),
          pl.BlockSpec((MBLK, NC * NS), lambda i: (i, 0)),
          pl.BlockSpec((1, d_hid), lambda i: (0, 0)),
          pl.BlockSpec((d_in, d_hid), lambda i: (0, 0)),
          pl.BlockSpec((d_hid, d_out), lambda i: (0, 0)),
      ],
      out_specs=[
          pl.BlockSpec((MBLK, d_out), lambda i: (i, 0)),
          pl.BlockSpec((MBLK, d_out // 2), lambda i: (i, 0)),
      ],
      out_shape=[
          jax.ShapeDtypeStruct((N, d_out), jnp.float32),
          jax.ShapeDtypeStruct((N, d_out // 2), jnp.int32),
      ],
  )(aggx, y, degp, b1, w1, w2)


def _final(agg2, h2p, degp, b2):
  d_out = h2p.shape[1]
  return pl.pallas_call(
      _final_body,
      grid=(N // MBLK,),
      in_specs=[
          pl.BlockSpec((MBLK, d_out), lambda i: (i, 0)),
          pl.BlockSpec((MBLK, d_out), lambda i: (i, 0)),
          pl.BlockSpec((MBLK, NC * NS), lambda i: (i, 0)),
          pl.BlockSpec((1, d_out), lambda i: (0, 0)),
      ],
      out_specs=pl.BlockSpec((MBLK, d_out), lambda i: (i, 0)),
      out_shape=jax.ShapeDtypeStruct((N, d_out), jnp.float32),
  )(agg2, h2p, degp, b2)


_agg_256 = _make_agg(256, 1)


@jax.jit
def kernel(x, edge_index, W1, b1, W2, b2):
  src = edge_index[0]
  dst = edge_index[1]

  degp = _deg_kernel(dst).T[:N]

  y, ypack = _scale(x, degp)
  aggx = _agg_256(src, dst, ypack)[:N]
  h2p, hpack = _mm12(aggx, y, degp, b1.reshape(1, -1), W1, W2)
  agg2 = _agg_256(src, dst, hpack)[:N]
  return _final(agg2, h2p, degp, b2.reshape(1, -1))
